# Initial kernel scaffold; baseline (speedup 1.0000x reference)
import jax
import jax.numpy as jnp
from jax.experimental import pallas as pl


def kernel(embed, embed_ele_indices):
    raise NotImplementedError("write your pallas kernel here")


# speedup vs baseline: 1.6266x; 1.6266x over previous
import jax
import jax.numpy as jnp
from jax.experimental import pallas as pl
from jax.experimental.pallas import tpu as pltpu

_FIELD_NUM = 26
_EMBED_DIM = 64
_EMBED_SIZE = _FIELD_NUM * _EMBED_DIM
_ROWS_PER_BLOCK = 1024
_IDX_PAD = 512


def _mask_mul_body(idx_ref, x_ref, o_ref, mask_ref):
    @pl.when(pl.program_id(0) == 0)
    def _build_mask():
        ids = idx_ref[...]
        pos = jax.lax.broadcasted_iota(jnp.int32, (_IDX_PAD, _EMBED_SIZE), 1)
        hit = jnp.any(ids == pos, axis=0, keepdims=True)
        mask_ref[...] = jnp.where(hit, 0.0, 1.0)

    o_ref[...] = x_ref[...] * mask_ref[...]


def kernel(embed, embed_ele_indices):
    B = embed.shape[0]
    x = embed.reshape(B, _EMBED_SIZE)
    idx = embed_ele_indices.astype(jnp.int32)
    pad = jnp.full((_IDX_PAD - idx.shape[0],), 2**30, dtype=jnp.int32)
    idx2 = jnp.concatenate([idx, pad]).reshape(_IDX_PAD, 1)

    out = pl.pallas_call(
        _mask_mul_body,
        grid=(B // _ROWS_PER_BLOCK,),
        in_specs=[
            pl.BlockSpec((_IDX_PAD, 1), lambda i: (0, 0)),
            pl.BlockSpec((_ROWS_PER_BLOCK, _EMBED_SIZE), lambda i: (i, 0)),
        ],
        out_specs=pl.BlockSpec((_ROWS_PER_BLOCK, _EMBED_SIZE), lambda i: (i, 0)),
        out_shape=jax.ShapeDtypeStruct((B, _EMBED_SIZE), jnp.float32),
        scratch_shapes=[pltpu.VMEM((1, _EMBED_SIZE), jnp.float32)],
    )(idx2, x)
    return out.reshape(B, _FIELD_NUM, _EMBED_DIM)


# speedup vs baseline: 1.6420x; 1.0095x over previous
import jax
import jax.numpy as jnp
from jax.experimental import pallas as pl
from jax.experimental.pallas import tpu as pltpu

_FIELD_NUM = 26
_EMBED_DIM = 64
_EMBED_SIZE = _FIELD_NUM * _EMBED_DIM
_ROWS_PER_BLOCK = 2048
_IDX_PAD = 512


def _mask_mul_body(idx_ref, x_ref, o_ref, mask_ref):
    @pl.when(pl.program_id(0) == 0)
    def _build_mask():
        ids = idx_ref[...]
        pos = jax.lax.broadcasted_iota(jnp.int32, (_IDX_PAD, _EMBED_SIZE), 1)
        hit = jnp.any(ids == pos, axis=0, keepdims=True)
        mask_ref[...] = jnp.where(hit, 0.0, 1.0)

    o_ref[...] = x_ref[...] * mask_ref[...]


def kernel(embed, embed_ele_indices):
    B = embed.shape[0]
    x = embed.reshape(B, _EMBED_SIZE)
    idx = embed_ele_indices.astype(jnp.int32)
    pad = jnp.full((_IDX_PAD - idx.shape[0],), 2**30, dtype=jnp.int32)
    idx2 = jnp.concatenate([idx, pad]).reshape(_IDX_PAD, 1)

    out = pl.pallas_call(
        _mask_mul_body,
        grid=(B // _ROWS_PER_BLOCK,),
        in_specs=[
            pl.BlockSpec((_IDX_PAD, 1), lambda i: (0, 0)),
            pl.BlockSpec((_ROWS_PER_BLOCK, _EMBED_SIZE), lambda i: (i, 0)),
        ],
        out_specs=pl.BlockSpec((_ROWS_PER_BLOCK, _EMBED_SIZE), lambda i: (i, 0)),
        out_shape=jax.ShapeDtypeStruct((B, _EMBED_SIZE), jnp.float32),
        scratch_shapes=[pltpu.VMEM((1, _EMBED_SIZE), jnp.float32)],
    )(idx2, x)
    return out.reshape(B, _FIELD_NUM, _EMBED_DIM)
